# baseline (speedup 1.0000x reference)
import functools

import jax
import jax.numpy as jnp
from jax import lax
from jax.experimental import pallas as pl
from jax.experimental.pallas import tpu as pltpu
from jax.experimental.pallas import tpu_sc as plsc

_LATENT = 128
_BATCH = 16384
_SC_ROWS = 8192
_TC_ROWS = _BATCH - _SC_ROWS
_NC = 2
_NS = 16
_L = 16
_NW = _NC * _NS
_ROWS_PER_W = _SC_ROWS // _NW
_CHUNK = 128
_NCHUNK = _ROWS_PER_W // _CHUNK
_CE = _CHUNK * _LATENT
_NBUF = 3
_GROUPS = _LATENT // _L

_TC_BLOCK = 1024


def _sc_permute(target_flat, permutation):
    mesh = plsc.VectorSubcoreMesh(
        core_axis_name="c", subcore_axis_name="s",
        num_cores=_NC, num_subcores=_NS)

    @functools.partial(
        pl.kernel,
        out_type=jax.ShapeDtypeStruct((_SC_ROWS * _LATENT,), jnp.float32),
        mesh=mesh,
        compiler_params=pltpu.CompilerParams(
            needs_layout_passes=False,
            disable_bounds_checks=True,
            disable_semaphore_checks=True,
            skip_device_barrier=True,
        ),
        scratch_types=[
            pltpu.VMEM((_LATENT,), jnp.int32),
            pltpu.VMEM((_CE,), jnp.float32),
            pltpu.VMEM((_CE,), jnp.float32),
            pltpu.VMEM((_CE,), jnp.float32),
            pltpu.VMEM((_CE,), jnp.float32),
            pltpu.VMEM((_CE,), jnp.float32),
            pltpu.VMEM((_CE,), jnp.float32),
            pltpu.SemaphoreType.DMA,
            pltpu.SemaphoreType.DMA,
            pltpu.SemaphoreType.DMA,
            pltpu.SemaphoreType.DMA,
            pltpu.SemaphoreType.DMA,
            pltpu.SemaphoreType.DMA,
        ],
    )
    def body(target_flat, perm_hbm, out_flat, perm_v,
             inb0, inb1, inb2, outb0, outb1, outb2,
             si0, si1, si2, so0, so1, so2):
        wid = lax.axis_index("s") * _NC + lax.axis_index("c")
        base = wid * _ROWS_PER_W * _LATENT
        inbs, outbs = [inb0, inb1, inb2], [outb0, outb1, outb2]
        sis, sos = [si0, si1, si2], [so0, so1, so2]

        in_h = [None] * _NBUF
        out_h = [None] * _NBUF
        for c in range(min(_NBUF, _NCHUNK)):
            in_h[c] = pltpu.async_copy(
                target_flat.at[pl.ds(base + c * _CE, _CE)], inbs[c], sis[c])
        pltpu.sync_copy(perm_hbm, perm_v)
        perm_vecs = tuple(perm_v[pl.ds(g * _L, _L)] for g in range(_GROUPS))

        for c in range(_NCHUNK):
            b = c % _NBUF
            in_h[b].wait()
            if out_h[b] is not None:
                out_h[b].wait()
            inb, outb = inbs[b], outbs[b]

            @plsc.parallel_loop(0, _CHUNK, 1, unroll=2)
            def _row(r, inb=inb, outb=outb):
                rb = r * _LATENT
                rbv = jnp.full((_L,), rb, dtype=jnp.int32)
                vals = [plsc.load_gather(inb, [perm_vecs[g] + rbv])
                        for g in range(_GROUPS)]
                for g in range(_GROUPS):
                    outb[pl.ds(rb + g * _L, _L)] = vals[g]
            out_h[b] = pltpu.async_copy(
                outb, out_flat.at[pl.ds(base + c * _CE, _CE)], sos[b])
            if c + _NBUF < _NCHUNK:
                in_h[b] = pltpu.async_copy(
                    target_flat.at[pl.ds(base + (c + _NBUF) * _CE, _CE)],
                    inbs[b], sis[b])

        for b in range(min(_NBUF, _NCHUNK)):
            if out_h[b] is not None:
                out_h[b].wait()

    return body(target_flat, permutation)


def _tc_body(perm_ref, x_ref, o_ref):
    pm = perm_ref[...]
    kk = lax.broadcasted_iota(jnp.int32, (_LATENT, _LATENT), 0)
    p = (kk == pm).astype(jnp.float32)
    o_ref[...] = jnp.dot(x_ref[...], p,
                         preferred_element_type=jnp.float32)


def _tc_permute(target_tc, permutation):
    grid = (_TC_ROWS // _TC_BLOCK,)
    return pl.pallas_call(
        _tc_body,
        grid=grid,
        in_specs=[
            pl.BlockSpec((1, _LATENT), lambda i: (0, 0)),
            pl.BlockSpec((_TC_BLOCK, _LATENT), lambda i: (i, 0)),
        ],
        out_specs=pl.BlockSpec((_TC_BLOCK, _LATENT), lambda i: (i, 0)),
        out_shape=jax.ShapeDtypeStruct((_TC_ROWS, _LATENT), jnp.float32),
    )(permutation.reshape(1, _LATENT), target_tc)


def kernel(target, permutation):
    sc_in = target[:_SC_ROWS].reshape(_SC_ROWS * _LATENT)
    sc_out = _sc_permute(sc_in, permutation)
    tc_out = _tc_permute(target[_SC_ROWS:], permutation)
    return jnp.concatenate(
        [sc_out.reshape(_SC_ROWS, _LATENT), tc_out], axis=0)


# speedup vs baseline: 1.3523x; 1.3523x over previous
import functools

import jax
import jax.numpy as jnp
from jax import lax
from jax.experimental import pallas as pl
from jax.experimental.pallas import tpu as pltpu
from jax.experimental.pallas import tpu_sc as plsc

_LATENT = 128
_BATCH = 16384
_NC = 2
_NS = 16
_L = 16
_NW = _NC * _NS
_ROWS_PER_W = _BATCH // _NW
_CHUNK = 128
_NCHUNK = _ROWS_PER_W // _CHUNK
_CE = _CHUNK * _LATENT
_NBUF = 3
_GROUPS = _LATENT // _L


def _sc_permute(target, permutation):
    mesh = plsc.VectorSubcoreMesh(
        core_axis_name="c", subcore_axis_name="s",
        num_cores=_NC, num_subcores=_NS)

    @functools.partial(
        pl.kernel,
        out_type=jax.ShapeDtypeStruct((_BATCH * _LATENT,), jnp.float32),
        mesh=mesh,
        compiler_params=pltpu.CompilerParams(
            needs_layout_passes=False,
            disable_bounds_checks=True,
            disable_semaphore_checks=True,
            skip_device_barrier=True,
        ),
        scratch_types=[
            pltpu.VMEM((_LATENT,), jnp.int32),
            pltpu.VMEM((_CE,), jnp.float32),
            pltpu.VMEM((_CE,), jnp.float32),
            pltpu.VMEM((_CE,), jnp.float32),
            pltpu.VMEM((_CE,), jnp.float32),
            pltpu.VMEM((_CE,), jnp.float32),
            pltpu.VMEM((_CE,), jnp.float32),
            pltpu.SemaphoreType.DMA,
            pltpu.SemaphoreType.DMA,
            pltpu.SemaphoreType.DMA,
            pltpu.SemaphoreType.DMA,
            pltpu.SemaphoreType.DMA,
            pltpu.SemaphoreType.DMA,
        ],
    )
    def body(target_flat, perm_hbm, out_flat, perm_v,
             inb0, inb1, inb2, outb0, outb1, outb2,
             si0, si1, si2, so0, so1, so2):
        wid = lax.axis_index("s") * _NC + lax.axis_index("c")
        base = wid * _ROWS_PER_W * _LATENT
        inbs, outbs = [inb0, inb1, inb2], [outb0, outb1, outb2]
        sis, sos = [si0, si1, si2], [so0, so1, so2]

        in_h = [None] * _NBUF
        out_h = [None] * _NBUF
        for c in range(min(_NBUF, _NCHUNK)):
            in_h[c] = pltpu.async_copy(
                target_flat.at[pl.ds(base + c * _CE, _CE)], inbs[c], sis[c])
        pltpu.sync_copy(perm_hbm, perm_v)
        perm_vecs = tuple(perm_v[pl.ds(g * _L, _L)] for g in range(_GROUPS))

        for c in range(_NCHUNK):
            b = c % _NBUF
            in_h[b].wait()
            if out_h[b] is not None:
                out_h[b].wait()
            inb, outb = inbs[b], outbs[b]

            @plsc.parallel_loop(0, _CHUNK, 1, unroll=4)
            def _row(r, inb=inb, outb=outb):
                rb = r * _LATENT
                rbv = jnp.full((_L,), rb, dtype=jnp.int32)
                vals = [plsc.load_gather(inb, [perm_vecs[g] + rbv])
                        for g in range(_GROUPS)]
                for g in range(_GROUPS):
                    outb[pl.ds(rb + g * _L, _L)] = vals[g]
            out_h[b] = pltpu.async_copy(
                outb, out_flat.at[pl.ds(base + c * _CE, _CE)], sos[b])
            if c + _NBUF < _NCHUNK:
                in_h[b] = pltpu.async_copy(
                    target_flat.at[pl.ds(base + (c + _NBUF) * _CE, _CE)],
                    inbs[b], sis[b])

        for b in range(min(_NBUF, _NCHUNK)):
            if out_h[b] is not None:
                out_h[b].wait()

    flat = body(target.reshape(_BATCH * _LATENT), permutation)
    return flat.reshape(_BATCH, _LATENT)


def kernel(target, permutation):
    return _sc_permute(target, permutation)


# speedup vs baseline: 1.3822x; 1.0221x over previous
import functools

import jax
import jax.numpy as jnp
from jax import lax
from jax.experimental import pallas as pl
from jax.experimental.pallas import tpu as pltpu
from jax.experimental.pallas import tpu_sc as plsc

_LATENT = 128
_BATCH = 16384
_NC = 2
_NS = 16
_L = 16
_NW = _NC * _NS
_ROWS_PER_W = _BATCH // _NW
_CHUNK = 128
_NCHUNK = _ROWS_PER_W // _CHUNK
_CE = _CHUNK * _LATENT
_NBUF = 3
_GROUPS = _LATENT // _L


def _sc_permute(target, permutation):
    mesh = plsc.VectorSubcoreMesh(
        core_axis_name="c", subcore_axis_name="s",
        num_cores=_NC, num_subcores=_NS)

    @functools.partial(
        pl.kernel,
        out_type=jax.ShapeDtypeStruct((_BATCH * _LATENT,), jnp.float32),
        mesh=mesh,
        compiler_params=pltpu.CompilerParams(
            needs_layout_passes=False,
            disable_bounds_checks=True,
            disable_semaphore_checks=True,
            skip_device_barrier=True,
        ),
        scratch_types=[
            pltpu.VMEM((_LATENT,), jnp.int32),
            pltpu.VMEM((_CE,), jnp.float32),
            pltpu.VMEM((_CE,), jnp.float32),
            pltpu.VMEM((_CE,), jnp.float32),
            pltpu.VMEM((_CE,), jnp.float32),
            pltpu.VMEM((_CE,), jnp.float32),
            pltpu.VMEM((_CE,), jnp.float32),
            pltpu.SemaphoreType.DMA,
            pltpu.SemaphoreType.DMA,
            pltpu.SemaphoreType.DMA,
            pltpu.SemaphoreType.DMA,
            pltpu.SemaphoreType.DMA,
            pltpu.SemaphoreType.DMA,
        ],
    )
    def body(target_flat, perm_hbm, out_flat, perm_v,
             inb0, inb1, inb2, outb0, outb1, outb2,
             si0, si1, si2, so0, so1, so2):
        wid = lax.axis_index("s") * _NC + lax.axis_index("c")
        base = wid * _ROWS_PER_W * _LATENT
        inbs, outbs = [inb0, inb1, inb2], [outb0, outb1, outb2]
        sis, sos = [si0, si1, si2], [so0, so1, so2]

        in_h = [None] * _NBUF
        out_h = [None] * _NBUF
        for c in range(min(_NBUF, _NCHUNK)):
            in_h[c] = pltpu.async_copy(
                target_flat.at[pl.ds(base + c * _CE, _CE)], inbs[c], sis[c])
        pltpu.sync_copy(perm_hbm, perm_v)
        perm_vecs = tuple(perm_v[pl.ds(g * _L, _L)] for g in range(_GROUPS))

        for c in range(_NCHUNK):
            b = c % _NBUF
            in_h[b].wait()
            if out_h[b] is not None:
                out_h[b].wait()
            inb, outb = inbs[b], outbs[b]

            @plsc.parallel_loop(0, _CHUNK, 1, unroll=2)
            def _row(r, inb=inb, outb=outb):
                rb = r * _LATENT
                rbv = jnp.full((_L,), rb, dtype=jnp.int32)
                vals = [plsc.load_gather(inb, [perm_vecs[g] + rbv])
                        for g in range(_GROUPS)]
                for g in range(_GROUPS):
                    outb[pl.ds(rb + g * _L, _L)] = vals[g]
            out_h[b] = pltpu.async_copy(
                outb, out_flat.at[pl.ds(base + c * _CE, _CE)], sos[b])
            if c + _NBUF < _NCHUNK:
                in_h[b] = pltpu.async_copy(
                    target_flat.at[pl.ds(base + (c + _NBUF) * _CE, _CE)],
                    inbs[b], sis[b])

        for b in range(min(_NBUF, _NCHUNK)):
            if out_h[b] is not None:
                out_h[b].wait()

    flat = body(target.reshape(_BATCH * _LATENT), permutation)
    return flat.reshape(_BATCH, _LATENT)


def kernel(target, permutation):
    return _sc_permute(target, permutation)
